# Initial kernel scaffold; baseline (speedup 1.0000x reference)
import jax
import jax.numpy as jnp
from jax.experimental import pallas as pl


def kernel(x, edge_index, W1l, b1l, W1r, b1r, att1, bias1, W2l, b2l, W2r, b2r, att2, bias2, Wout, bout):
    raise NotImplementedError("write your pallas kernel here")


# speedup vs baseline: 7.4025x; 7.4025x over previous
import functools

import jax
import jax.numpy as jnp
from jax import lax
from jax.experimental import pallas as pl
from jax.experimental.pallas import tpu as pltpu
from jax.experimental.pallas import tpu_sc as plsc

NC = 2
NS = 16
L = 16
K = 80
ZROWS = 104


def _edge_pass(xl, xr, src, dst, att):
    n, d = xl.shape
    e = src.shape[0]
    nw = NC * NS
    ew = e // nw
    nchunk = ew // K
    rpt = (n // NS) // 8 * 8
    tail = n - NS * rpt
    nj = d // L

    mesh = plsc.VectorSubcoreMesh(core_axis_name="c", subcore_axis_name="s",
                                  num_cores=NC, num_subcores=NS)

    @functools.partial(
        pl.kernel,
        out_type=(
            jax.ShapeDtypeStruct((NC, n, d), jnp.float32),
            jax.ShapeDtypeStruct((nw, 1, n), jnp.float32),
        ),
        mesh=mesh,
        scratch_types=[
            pltpu.VMEM_SHARED((n, d), jnp.float32),
            pltpu.VMEM((n + L,), jnp.float32),
            pltpu.VMEM((K,), jnp.int32),
            pltpu.VMEM((K,), jnp.int32),
            pltpu.VMEM((K, d), jnp.float32),
            pltpu.VMEM((K, d), jnp.float32),
            pltpu.VMEM((K, d), jnp.float32),
            pltpu.VMEM((d,), jnp.float32),
            pltpu.SemaphoreType.DMA,
            pltpu.SemaphoreType.DMA,
        ],
    )
    def body(xl_h, xr_h, src_h, dst_h, att_h, num_h, z_h,
             num_sh, ztile, src_v, dst_v, xlv, xrv, msgv, attv,
             sem1, sem2):
        cid = lax.axis_index("c")
        sid = lax.axis_index("s")
        wid = sid * NC + cid

        def zrow(i, carry):
            for j in range(nj):
                msgv[i, pl.ds(L * j, L)] = jnp.zeros((L,), jnp.float32)
            return carry
        lax.fori_loop(0, K, zrow, 0)

        def zz(i, carry):
            ztile[pl.ds(i * L, L)] = jnp.zeros((L,), jnp.float32)
            return carry
        lax.fori_loop(0, (n + L) // L, zz, 0)
        row0 = sid * rpt
        for t in range(rpt // K):
            pltpu.sync_copy(msgv, num_sh.at[pl.ds(row0 + K * t, K)])
        rem = rpt - (rpt // K) * K
        if rem:
            pltpu.sync_copy(msgv.at[pl.ds(0, rem)],
                            num_sh.at[pl.ds(row0 + (rpt // K) * K, rem)])

        @pl.when(sid == NS - 1)
        def _zero_tail():
            pltpu.sync_copy(msgv.at[pl.ds(0, tail)],
                            num_sh.at[pl.ds(NS * rpt, tail)])
        plsc.subcore_barrier()

        pltpu.sync_copy(att_h, attv)
        att_regs = [attv[pl.ds(L * j, L)] for j in range(nj)]
        lanes = lax.iota(jnp.int32, L)
        bfly = [jnp.bitwise_xor(lanes, sh) for sh in (8, 4, 2, 1)]
        fzero = jnp.zeros((L,), jnp.float32)

        def chunk(ci, carry):
            base = wid * ew + ci * K
            pltpu.sync_copy(src_h.at[pl.ds(base, K)], src_v)
            pltpu.sync_copy(dst_h.at[pl.ds(base, K)], dst_v)
            pltpu.async_copy(xl_h.at[src_v], xlv, sem1).wait()
            pltpu.async_copy(xr_h.at[dst_v], xrv, sem2).wait()

            def group(g, c2):
                didx = dst_v[pl.ds(g * L, L)]
                for j in range(L):
                    k = g * L + j
                    rows = []
                    acc = jnp.zeros((L,), jnp.float32)
                    for jj in range(nj):
                        a = xlv[k, pl.ds(L * jj, L)]
                        b = xrv[k, pl.ds(L * jj, L)]
                        t = a + b
                        h = jnp.maximum(t, 0.2 * t)
                        acc = acc + h * att_regs[jj]
                        rows.append(a)
                    for idx in bfly:
                        acc = acc + acc.at[idx].get(mode="promise_in_bounds")
                    p = jnp.exp(acc)
                    for jj in range(nj):
                        msgv[k, pl.ds(L * jj, L)] = rows[jj] * p
                    dk = didx[j]
                    zbase = (dk // L) * L
                    posv = jnp.full((L,), dk - zbase, jnp.int32)
                    padd = jnp.where(lanes == posv, p, fzero)
                    ztile[pl.ds(zbase, L)] = ztile[pl.ds(zbase, L)] + padd
                return c2
            lax.fori_loop(0, K // L, group, 0)

            pltpu.sync_copy(msgv, num_sh.at[dst_v], add=True)
            return carry
        lax.fori_loop(0, nchunk, chunk, 0)
        plsc.subcore_barrier()

        pltpu.sync_copy(num_sh.at[pl.ds(row0, rpt)],
                        num_h.at[cid, pl.ds(row0, rpt)])
        pltpu.sync_copy(ztile.at[pl.ds(0, n)], z_h.at[wid, 0])

        @pl.when(sid == NS - 1)
        def _read_tail():
            pltpu.sync_copy(num_sh.at[pl.ds(NS * rpt, tail)],
                            num_h.at[cid, pl.ds(NS * rpt, tail)])

    return body(xl, xr, src, dst, att)


_ROWS = 1000


def _proj2(x, Wl, bl, Wr, br):
    n, d = x.shape

    def body(x_ref, wl_ref, bl_ref, wr_ref, br_ref, xl_ref, xr_ref):
        xx = x_ref[...]
        xl_ref[...] = jnp.dot(xx, wl_ref[...],
                              precision=lax.Precision.HIGHEST) + bl_ref[...]
        xr_ref[...] = jnp.dot(xx, wr_ref[...],
                              precision=lax.Precision.HIGHEST) + br_ref[...]

    return pl.pallas_call(
        body,
        grid=(n // _ROWS,),
        in_specs=[
            pl.BlockSpec((_ROWS, d), lambda i: (i, 0)),
            pl.BlockSpec((d, d), lambda i: (0, 0)),
            pl.BlockSpec((1, d), lambda i: (0, 0)),
            pl.BlockSpec((d, d), lambda i: (0, 0)),
            pl.BlockSpec((1, d), lambda i: (0, 0)),
        ],
        out_specs=[
            pl.BlockSpec((_ROWS, d), lambda i: (i, 0)),
            pl.BlockSpec((_ROWS, d), lambda i: (i, 0)),
        ],
        out_shape=[jax.ShapeDtypeStruct((n, d), jnp.float32)] * 2,
    )(x, Wl, bl, Wr, br)


def _combine_proj2(num, z, bias, Wl, bl, Wr, br):
    _, n, d = num.shape
    nw = z.shape[1]

    def body(np_ref, zp_ref, bias_ref, wl_ref, bl_ref, wr_ref, br_ref,
             xl_ref, xr_ref):
        acc = np_ref[0] + np_ref[1]
        zz = jnp.sum(zp_ref[...], axis=1, keepdims=True)
        h = acc / (zz + 1e-30) + bias_ref[...]
        h = jnp.maximum(h, 0.0)
        xl_ref[...] = jnp.dot(h, wl_ref[...],
                              precision=lax.Precision.HIGHEST) + bl_ref[...]
        xr_ref[...] = jnp.dot(h, wr_ref[...],
                              precision=lax.Precision.HIGHEST) + br_ref[...]

    return pl.pallas_call(
        body,
        grid=(n // _ROWS,),
        in_specs=[
            pl.BlockSpec((NC, _ROWS, d), lambda i: (0, i, 0)),
            pl.BlockSpec((_ROWS, nw), lambda i: (i, 0)),
            pl.BlockSpec((1, d), lambda i: (0, 0)),
            pl.BlockSpec((d, d), lambda i: (0, 0)),
            pl.BlockSpec((1, d), lambda i: (0, 0)),
            pl.BlockSpec((d, d), lambda i: (0, 0)),
            pl.BlockSpec((1, d), lambda i: (0, 0)),
        ],
        out_specs=[
            pl.BlockSpec((_ROWS, d), lambda i: (i, 0)),
            pl.BlockSpec((_ROWS, d), lambda i: (i, 0)),
        ],
        out_shape=[jax.ShapeDtypeStruct((n, d), jnp.float32)] * 2,
    )(num, z, bias, Wl, bl, Wr, br)


def _combine_out(num, z, bias, W, b):
    _, n, d = num.shape
    nw = z.shape[1]
    dout = W.shape[1]

    def body(np_ref, zp_ref, bias_ref, w_ref, b_ref, o_ref):
        acc = np_ref[0] + np_ref[1]
        zz = jnp.sum(zp_ref[...], axis=1, keepdims=True)
        h = acc / (zz + 1e-30) + bias_ref[...]
        o_ref[...] = jnp.dot(h, w_ref[...],
                             precision=lax.Precision.HIGHEST) + b_ref[...]

    return pl.pallas_call(
        body,
        grid=(n // _ROWS,),
        in_specs=[
            pl.BlockSpec((NC, _ROWS, d), lambda i: (0, i, 0)),
            pl.BlockSpec((_ROWS, nw), lambda i: (i, 0)),
            pl.BlockSpec((1, d), lambda i: (0, 0)),
            pl.BlockSpec((d, dout), lambda i: (0, 0)),
            pl.BlockSpec((1, dout), lambda i: (0, 0)),
        ],
        out_specs=pl.BlockSpec((_ROWS, dout), lambda i: (i, 0)),
        out_shape=jax.ShapeDtypeStruct((n, dout), jnp.float32),
    )(num, z, bias, W, b)


def kernel(x, edge_index, W1l, b1l, W1r, b1r, att1, bias1,
           W2l, b2l, W2r, b2r, att2, bias2, Wout, bout):
    src = edge_index[0]
    dst = edge_index[1]
    xl1, xr1 = _proj2(x, W1l, b1l.reshape(1, -1), W1r, b1r.reshape(1, -1))
    num1, z1 = _edge_pass(xl1, xr1, src, dst, att1.reshape(-1))
    z1t = z1.reshape(z1.shape[0], -1).T
    xl2, xr2 = _combine_proj2(num1, z1t, bias1.reshape(1, -1),
                              W2l, b2l.reshape(1, -1),
                              W2r, b2r.reshape(1, -1))
    num2, z2 = _edge_pass(xl2, xr2, src, dst, att2.reshape(-1))
    z2t = z2.reshape(z2.shape[0], -1).T
    return _combine_out(num2, z2t, bias2.reshape(1, -1),
                        Wout, bout.reshape(1, -1))
